# Initial kernel scaffold; baseline (speedup 1.0000x reference)
import jax
import jax.numpy as jnp
from jax.experimental import pallas as pl


def kernel(edge_index, edge_type, node_emb, W1, root1, bias1, W2, root2, bias2):
    raise NotImplementedError("write your pallas kernel here")


# speedup vs baseline: 6.9373x; 6.9373x over previous
import functools

import jax
import jax.numpy as jnp
from jax import lax
from jax.experimental import pallas as pl
from jax.experimental.pallas import tpu as pltpu
from jax.experimental.pallas import tpu_sc as plsc

N = 10000
R = 30
H = 200
B = 5
D = H // B
HP = 256
HH = HP // 2
NB = 1000
NT = 16
NC = 2
K = 128
GC = 32
NR = N * R
CNTM = NR + 32
N2 = 10240
NROWS_T = N2 // NT


def _tf1_body(x_ref, w_ref, root_ref, bias_ref, y_ref, base_ref, xb_ref):
    r = pl.program_id(1)

    @pl.when(r == 0)
    def _():
        xb_ref[...] = x_ref[...].astype(jnp.bfloat16)
        base_ref[...] = (
            jnp.dot(xb_ref[...], root_ref[...], preferred_element_type=jnp.float32)
            + bias_ref[...]
        )

    y = jnp.dot(xb_ref[...], w_ref[0], preferred_element_type=jnp.float32)
    y_ref[0, 0] = y[:, :HH]
    y_ref[1, 0] = y[:, HH:]


def _transform1(x, wb, rootb, bias2d):
    return pl.pallas_call(
        _tf1_body,
        grid=(N // NB, R),
        in_specs=[
            pl.BlockSpec((NB, H), lambda i, r: (i, 0)),
            pl.BlockSpec((1, H, HP), lambda i, r: (r, 0, 0)),
            pl.BlockSpec((H, H), lambda i, r: (0, 0)),
            pl.BlockSpec((1, H), lambda i, r: (0, 0)),
        ],
        out_specs=[
            pl.BlockSpec((2, 1, NB, HH), lambda i, r: (0, r, i, 0)),
            pl.BlockSpec((NB, H), lambda i, r: (i, 0)),
        ],
        out_shape=[
            jax.ShapeDtypeStruct((2, R, N, HH), jnp.float32),
            jax.ShapeDtypeStruct((N, H), jnp.float32),
        ],
        scratch_shapes=[pltpu.VMEM((NB, H), jnp.bfloat16)],
    )(x, wb, rootb, bias2d)


def _tf2_body(agg_ref, bin_ref, w_ref, root_ref, bias_ref, y_ref, base_ref,
              xb_ref):
    r = pl.program_id(1)

    @pl.when(r == 0)
    def _():
        x = jnp.concatenate([agg_ref[0], agg_ref[1, :, :H - HH]], axis=1)
        x = jnp.maximum(x + bin_ref[...], 0.0)
        xb_ref[...] = x.astype(jnp.bfloat16)
        base_ref[...] = (
            jnp.dot(xb_ref[...], root_ref[...], preferred_element_type=jnp.float32)
            + bias_ref[...]
        )

    y = jnp.dot(xb_ref[...], w_ref[0], preferred_element_type=jnp.float32)
    y_ref[0, 0] = y[:, :HH]
    y_ref[1, 0] = y[:, HH:]


def _transform2(agg, base_in, wb, rootb, bias2d):
    return pl.pallas_call(
        _tf2_body,
        grid=(N // NB, R),
        in_specs=[
            pl.BlockSpec((2, NB, HH), lambda i, r: (0, i, 0)),
            pl.BlockSpec((NB, H), lambda i, r: (i, 0)),
            pl.BlockSpec((1, H, HP), lambda i, r: (r, 0, 0)),
            pl.BlockSpec((H, H), lambda i, r: (0, 0)),
            pl.BlockSpec((1, H), lambda i, r: (0, 0)),
        ],
        out_specs=[
            pl.BlockSpec((2, 1, NB, HH), lambda i, r: (0, r, i, 0)),
            pl.BlockSpec((NB, H), lambda i, r: (i, 0)),
        ],
        out_shape=[
            jax.ShapeDtypeStruct((2, R, N, HH), jnp.float32),
            jax.ShapeDtypeStruct((N, H), jnp.float32),
        ],
        scratch_shapes=[pltpu.VMEM((NB, H), jnp.bfloat16)],
    )(agg, base_in, wb, rootb, bias2d)


def _fin_body(agg_ref, base_ref, o_ref):
    o_ref[...] = (
        jnp.concatenate([agg_ref[0], agg_ref[1, :, :H - HH]], axis=1)
        + base_ref[...]
    )


def _final(agg, base):
    return pl.pallas_call(
        _fin_body,
        grid=(N // NB,),
        in_specs=[
            pl.BlockSpec((2, NB, HH), lambda i: (0, i, 0)),
            pl.BlockSpec((NB, H), lambda i: (i, 0)),
        ],
        out_specs=pl.BlockSpec((NB, H), lambda i: (i, 0)),
        out_shape=jax.ShapeDtypeStruct((N, H), jnp.float32),
    )(agg, base)


def _sc_mesh():
    return plsc.VectorSubcoreMesh(core_axis_name="c", subcore_axis_name="s")


def _make_norm_kernel(ct, e_real):
    @functools.partial(
        pl.kernel, mesh=_sc_mesh(),
        out_type=jax.ShapeDtypeStruct((NT, ct, K), jnp.float32),
        scratch_types=[
            pltpu.VMEM_SHARED((CNTM,), jnp.float32),
            pltpu.VMEM((ct, K), jnp.int32),
            pltpu.VMEM((ct, K), jnp.float32),
            pltpu.VMEM((K,), jnp.float32),
            pltpu.VMEM((CNTM // NT,), jnp.float32),
            pltpu.SemaphoreType.DMA,
        ],
    )
    def _norm(comp_hbm, out_hbm, cnt_sh, comp_v, val_v, ones_v, zc_v, sem):
        c = lax.axis_index("c")
        s = lax.axis_index("s")

        @pl.when(c == 0)
        def _():
            zsl = CNTM // NT

            def zfill(i, carry):
                zc_v[pl.ds(i * 16, 16)] = jnp.zeros((16,), jnp.float32)
                return carry

            lax.fori_loop(0, zsl // 16, zfill, 0)
            pltpu.sync_copy(zc_v, cnt_sh.at[pl.ds(s * zsl, zsl)])
            pltpu.sync_copy(comp_hbm.at[s], comp_v)
            for u in range(K // 16):
                ones_v[pl.ds(u * 16, 16)] = jnp.full((16,), 1.0, jnp.float32)
            plsc.subcore_barrier()

            def p1(j, carry):
                pltpu.sync_copy(ones_v, cnt_sh.at[comp_v.at[j]], add=True)
                return carry

            lax.fori_loop(0, ct, p1, 0)
            plsc.subcore_barrier()

            et_per_tile = ct * K

            def p2(j, carry):
                pltpu.async_copy(cnt_sh.at[comp_v.at[j]], val_v.at[j],
                                 sem).wait()
                for u in range(K // 16):
                    cnt = val_v[j, pl.ds(u * 16, 16)]
                    pos = (s * et_per_tile + j * K + u * 16
                           + lax.iota(jnp.int32, 16))
                    nrm = 1.0 / jnp.maximum(cnt, 1.0)
                    val_v[j, pl.ds(u * 16, 16)] = jnp.where(
                        pos < e_real, nrm, 0.0)
                return carry

            lax.fori_loop(0, ct, p2, 0)
            pltpu.sync_copy(val_v, out_hbm.at[s])

    return _norm


def _make_agg_kernel(ct):
    ng = ct // GC
    npair = GC // 2

    @functools.partial(
        pl.kernel, mesh=_sc_mesh(),
        out_type=jax.ShapeDtypeStruct((NC, N2, HH), jnp.float32),
        scratch_types=[
            pltpu.VMEM_SHARED((N2, HH), jnp.float32),
            pltpu.VMEM((GC, K), jnp.int32),
            pltpu.VMEM((GC, K), jnp.int32),
            pltpu.VMEM((GC, K), jnp.float32),
            pltpu.VMEM((K, HH), jnp.float32),
            pltpu.VMEM((K, HH), jnp.float32),
            pltpu.SemaphoreType.DMA,
            pltpu.SemaphoreType.DMA,
        ],
    )
    def _agg(y_hbm, gidx_hbm, dst_hbm, nrm_hbm, out_hbm,
             agg_sh, gix_v, dst_v, nrm_v, rows0, rows1, gsem0, gsem1):
        c = lax.axis_index("c")
        s = lax.axis_index("s")
        rs = s * NROWS_T

        def zfill(i, carry):
            for u in range(HH // 16):
                rows0[i, pl.ds(u * 16, 16)] = jnp.zeros((16,), jnp.float32)
            return carry

        lax.fori_loop(0, K, zfill, 0)
        for t in range(NROWS_T // K):
            pltpu.sync_copy(rows0, agg_sh.at[pl.ds(rs + t * K, K)])
        plsc.subcore_barrier()

        coff = c * (R * N)

        def scale(rows, jj):
            def body(q, c2):
                nvec = nrm_v[jj, pl.ds(q * 16, 16)]
                for i in range(16):
                    nv = nvec[i]
                    k = q * 16 + i
                    for u in range(HH // 16):
                        rows[k, pl.ds(u * 16, 16)] = (
                            rows[k, pl.ds(u * 16, 16)] * nv)
                return c2

            lax.fori_loop(0, K // 16, body, 0)

        def group(g, carry):
            pltpu.sync_copy(gidx_hbm.at[s, pl.ds(g * GC, GC)], gix_v)
            pltpu.sync_copy(dst_hbm.at[s, pl.ds(g * GC, GC)], dst_v)
            pltpu.sync_copy(nrm_hbm.at[s, pl.ds(g * GC, GC)], nrm_v)

            def addoff(j, c2):
                for u in range(K // 16):
                    gix_v[j, pl.ds(u * 16, 16)] = (
                        gix_v[j, pl.ds(u * 16, 16)] + coff)
                return c2

            lax.fori_loop(0, GC, addoff, 0)
            pltpu.async_copy(y_hbm.at[gix_v.at[0]], rows0, gsem0)

            def pair(p, c2):
                j0 = 2 * p
                j1 = 2 * p + 1
                pltpu.async_copy(y_hbm.at[gix_v.at[j1]], rows1, gsem1)
                pltpu.make_async_copy(y_hbm.at[gix_v.at[j0]], rows0,
                                      gsem0).wait()
                scale(rows0, j0)
                pltpu.sync_copy(rows0, agg_sh.at[dst_v.at[j0]], add=True)

                @pl.when(p < npair - 1)
                def _():
                    pltpu.async_copy(y_hbm.at[gix_v.at[j0 + 2]], rows0,
                                     gsem0)

                pltpu.make_async_copy(y_hbm.at[gix_v.at[j1]], rows1,
                                      gsem1).wait()
                scale(rows1, j1)
                pltpu.sync_copy(rows1, agg_sh.at[dst_v.at[j1]], add=True)
                return c2

            lax.fori_loop(0, npair, pair, 0)
            return carry

        lax.fori_loop(0, ng, group, 0)
        plsc.subcore_barrier()
        for t in range(NROWS_T // K):
            pltpu.sync_copy(agg_sh.at[pl.ds(rs + t * K, K)], rows0)
            pltpu.sync_copy(rows0, out_hbm.at[c, pl.ds(rs + t * K, K)])

    return _agg


def kernel(edge_index, edge_type, node_emb, W1, root1, bias1, W2, root2,
           bias2):
    e_real = edge_index.shape[1]
    ct = ((e_real + NT * K * GC - 1) // (NT * K * GC)) * GC
    ep = ct * NT * K
    pad = ep - e_real

    src = edge_index[0].astype(jnp.int32)
    dst = edge_index[1].astype(jnp.int32)
    et = edge_type.astype(jnp.int32)
    eidx = jnp.arange(ep, dtype=jnp.int32)
    valid = eidx < e_real
    srcp = jnp.concatenate([src, jnp.zeros((pad,), jnp.int32)])
    dstp = jnp.concatenate([dst, jnp.zeros((pad,), jnp.int32)])
    etp = jnp.concatenate([et, jnp.zeros((pad,), jnp.int32)])

    gidx = jnp.where(valid, etp * N + srcp, eidx % N)
    comp = jnp.where(valid, dstp * R + etp, NR + (eidx % 32))

    gidx3 = gidx.reshape(NT, ct, K)
    dst3 = dstp.reshape(NT, ct, K)
    comp3 = comp.reshape(NT, ct, K)

    norm3 = _make_norm_kernel(ct, e_real)(comp3)

    def blockdiag(w):
        bd = jnp.zeros((R, H, HP), jnp.float32)
        for b in range(B):
            bd = bd.at[:, b * D:(b + 1) * D, b * D:(b + 1) * D].set(w[:, b])
        return bd.astype(jnp.bfloat16)

    w1b = blockdiag(W1)
    w2b = blockdiag(W2)
    r1b = root1.astype(jnp.bfloat16)
    r2b = root2.astype(jnp.bfloat16)
    b1 = bias1.reshape(1, H)
    b2 = bias2.reshape(1, H)

    agg_kernel = _make_agg_kernel(ct)

    y1, base1 = _transform1(node_emb, w1b, r1b, b1)
    agg1 = agg_kernel(y1.reshape(NC * R * N, HH), gidx3, dst3, norm3)
    y2, base2 = _transform2(agg1, base1, w2b, r2b, b2)
    agg2 = agg_kernel(y2.reshape(NC * R * N, HH), gidx3, dst3, norm3)
    return _final(agg2, base2)
